# baseline (speedup 1.0000x reference)
import functools

import jax
import jax.numpy as jnp
from jax import lax
from jax.experimental import pallas as pl
from jax.experimental.pallas import tpu as pltpu
from jax.experimental.pallas import tpu_sc as plsc

N = 10000
E = 320000
NC = 2
NS = 16
K = 128
CH = 80
HCH = CH // 2
E_PAD = NC * NS * CH * K
AGG_ROWS = 10112
TROWS = AGG_ROWS // NS
BR = 1000


def _sc_aggregate(g, src_r, dst_r, feat):
    mesh = plsc.VectorSubcoreMesh(
        core_axis_name="c", subcore_axis_name="s", num_cores=NC, num_subcores=NS
    )

    @functools.partial(
        pl.kernel,
        out_type=jax.ShapeDtypeStruct((NC, AGG_ROWS, feat), jnp.float32),
        mesh=mesh,
        scratch_types=[
            pltpu.VMEM((HCH, K), jnp.int32),
            pltpu.VMEM((HCH, K), jnp.int32),
            pltpu.VMEM((2, K, feat), jnp.float32),
            pltpu.VMEM_SHARED((AGG_ROWS, feat), jnp.float32),
            pltpu.SemaphoreType.DMA,
            pltpu.SemaphoreType.DMA,
        ],
    )
    def agg_kernel(
        g_hbm, src_hbm, dst_hbm, out_hbm, idx_s, idx_d, buf, acc, sem_g, sem_s
    ):
        cid = lax.axis_index("c")
        sid = lax.axis_index("s")

        zeros = jnp.zeros((16,), jnp.float32)

        def zero_body(i, _):
            r = i // (feat // 16)
            c = i % (feat // 16)
            buf[0, r, pl.ds(c * 16, 16)] = zeros
            return 0

        lax.fori_loop(0, K * (feat // 16), zero_body, 0)
        off = 0
        while off < TROWS:
            rows = min(K, TROWS - off)
            pltpu.sync_copy(
                buf.at[0, pl.ds(0, rows)], acc.at[pl.ds(sid * TROWS + off, rows)]
            )
            off += rows
        plsc.subcore_barrier()

        for half in range(CH // HCH):
            pltpu.sync_copy(
                src_hbm.at[cid, sid, pl.ds(half * HCH, HCH)], idx_s
            )
            pltpu.sync_copy(
                dst_hbm.at[cid, sid, pl.ds(half * HCH, HCH)], idx_d
            )
            pltpu.async_copy(g_hbm.at[pl.ds(0, K)], buf.at[0], sem_g)

            def chunk_body(j, _):
                p = j % 2
                q = 1 - p
                pltpu.make_async_copy(
                    g_hbm.at[pl.ds(0, K)], buf.at[p], sem_g
                ).wait()

                @pl.when(j >= 1)
                def _():
                    pltpu.make_async_copy(
                        buf.at[q], acc.at[idx_d.at[j - 1]], sem_s
                    ).wait()

                @pl.when(j + 1 < HCH)
                def _():
                    pltpu.async_copy(g_hbm.at[pl.ds(0, K)], buf.at[q], sem_g)

                pltpu.async_copy(buf.at[p], acc.at[idx_d.at[j]], sem_s, add=True)
                return 0

            lax.fori_loop(0, HCH, chunk_body, 0)
            last = (HCH - 1) % 2
            pltpu.make_async_copy(
                buf.at[last], acc.at[idx_d.at[HCH - 1]], sem_s
            ).wait()
        plsc.subcore_barrier()

        off = 0
        while off < TROWS:
            rows = min(K, TROWS - off)
            pltpu.sync_copy(
                acc.at[pl.ds(sid * TROWS + off, rows)],
                out_hbm.at[cid, pl.ds(sid * TROWS + off, rows)],
            )
            off += rows

    return agg_kernel(g, src_r, dst_r)


def _mm_body(x_ref, w_ref, o_ref):
    o_ref[...] = jnp.dot(x_ref[...], w_ref[...], preferred_element_type=jnp.float32)


def _matmul(x, w):
    n, fin = x.shape
    fout = w.shape[1]
    return pl.pallas_call(
        _mm_body,
        grid=(n // BR,),
        in_specs=[
            pl.BlockSpec((BR, fin), lambda i: (i, 0)),
            pl.BlockSpec((fin, fout), lambda i: (0, 0)),
        ],
        out_specs=pl.BlockSpec((BR, fout), lambda i: (i, 0)),
        out_shape=jax.ShapeDtypeStruct((n, fout), jnp.float32),
    )(x, w)


def _combine_mm_body(g_ref, a0_ref, a1_ref, b_ref, w_ref, o_ref):
    h = g_ref[...] + a0_ref[0] + a1_ref[0] + b_ref[...]
    h = jnp.maximum(h, 0.0)
    o_ref[...] = jnp.dot(h, w_ref[...], preferred_element_type=jnp.float32)


def _combine_matmul(g, agg, b, w):
    n, fin = g.shape
    fout = w.shape[1]
    return pl.pallas_call(
        _combine_mm_body,
        grid=(n // BR,),
        in_specs=[
            pl.BlockSpec((BR, fin), lambda i: (i, 0)),
            pl.BlockSpec((1, BR, fin), lambda i: (0, i, 0)),
            pl.BlockSpec((1, BR, fin), lambda i: (1, i, 0)),
            pl.BlockSpec((1, fin), lambda i: (0, 0)),
            pl.BlockSpec((fin, fout), lambda i: (0, 0)),
        ],
        out_specs=pl.BlockSpec((BR, fout), lambda i: (i, 0)),
        out_shape=jax.ShapeDtypeStruct((n, fout), jnp.float32),
    )(g, agg, agg, b.reshape(1, fin), w)


def _combine_relu_body(g_ref, a0_ref, a1_ref, b_ref, o_ref):
    o_ref[...] = jnp.maximum(g_ref[...] + a0_ref[0] + a1_ref[0] + b_ref[...], 0.0)


def _combine_relu(g, agg, b):
    n, f = g.shape
    return pl.pallas_call(
        _combine_relu_body,
        grid=(n // BR,),
        in_specs=[
            pl.BlockSpec((BR, f), lambda i: (i, 0)),
            pl.BlockSpec((1, BR, f), lambda i: (0, i, 0)),
            pl.BlockSpec((1, BR, f), lambda i: (1, i, 0)),
            pl.BlockSpec((1, f), lambda i: (0, 0)),
        ],
        out_specs=pl.BlockSpec((BR, f), lambda i: (i, 0)),
        out_shape=jax.ShapeDtypeStruct((n, f), jnp.float32),
    )(g, agg, agg, b.reshape(1, f))


def _final_mm_body(h_ref, a0_ref, a1_ref, w_ref, b_ref, o_ref):
    hh = h_ref[...] + a0_ref[0] + a1_ref[0]
    o_ref[...] = (
        jnp.dot(hh, w_ref[...], preferred_element_type=jnp.float32) + b_ref[...]
    )


def _final_matmul(h, agg, w, b):
    n, fin = h.shape
    fout = w.shape[1]
    return pl.pallas_call(
        _final_mm_body,
        grid=(n // BR,),
        in_specs=[
            pl.BlockSpec((BR, fin), lambda i: (i, 0)),
            pl.BlockSpec((1, BR, fin), lambda i: (0, i, 0)),
            pl.BlockSpec((1, BR, fin), lambda i: (1, i, 0)),
            pl.BlockSpec((fin, fout), lambda i: (0, 0)),
            pl.BlockSpec((1, fout), lambda i: (0, 0)),
        ],
        out_specs=pl.BlockSpec((BR, fout), lambda i: (i, 0)),
        out_shape=jax.ShapeDtypeStruct((n, fout), jnp.float32),
    )(h, agg, agg, w, b.reshape(1, fout))


def kernel(x, edge_index, W1, b1, W2, b2, W3, b3, W4, b4):
    src = edge_index[0]
    dst = edge_index[1]
    pad = E_PAD - E
    pad_src = (jnp.arange(pad, dtype=jnp.int32) * 997) % N
    pad_dst = N + jnp.arange(pad, dtype=jnp.int32) % (AGG_ROWS - N)
    src_r = jnp.concatenate([src, pad_src]).reshape(NC, NS, CH, K)
    dst_r = jnp.concatenate([dst, pad_dst]).reshape(NC, NS, CH, K)

    g = _matmul(x, W1)
    agg = _sc_aggregate(g, src_r, dst_r, 128)
    g = _combine_matmul(g, agg, b1, W2)
    agg = _sc_aggregate(g, src_r, dst_r, 128)
    g = _combine_matmul(g, agg, b2, W3)
    agg = _sc_aggregate(g, src_r, dst_r, 128)
    h = _combine_relu(g, agg, b3)
    agg = _sc_aggregate(h, src_r, dst_r, 128)
    return _final_matmul(h, agg, W4, b4)


# speedup vs baseline: 2.3285x; 2.3285x over previous
import functools

import jax
import jax.numpy as jnp
from jax import lax
from jax.experimental import pallas as pl
from jax.experimental.pallas import tpu as pltpu
from jax.experimental.pallas import tpu_sc as plsc

N = 10000
E = 320000
NC = 2
NS = 16
K = 64
CH = 160
HCH = CH // 4
NBUF = 4
E_PAD = NC * NS * CH * K
AGG_ROWS = 10112
TROWS = AGG_ROWS // NS
BR = 1000


def _sc_aggregate(g, src_r, dst_r, feat):
    mesh = plsc.VectorSubcoreMesh(
        core_axis_name="c", subcore_axis_name="s", num_cores=NC, num_subcores=NS
    )

    @functools.partial(
        pl.kernel,
        out_type=jax.ShapeDtypeStruct((NC, AGG_ROWS, feat), jnp.float32),
        mesh=mesh,
        scratch_types=[
            pltpu.VMEM((HCH, K), jnp.int32),
            pltpu.VMEM((HCH, K), jnp.int32),
            pltpu.VMEM((NBUF, K, feat), jnp.float32),
            pltpu.VMEM_SHARED((AGG_ROWS, feat), jnp.float32),
            pltpu.SemaphoreType.DMA,
            pltpu.SemaphoreType.DMA,
        ],
    )
    def agg_kernel(
        g_hbm, src_hbm, dst_hbm, out_hbm, idx_s, idx_d, buf, acc, sem_g, sem_s
    ):
        cid = lax.axis_index("c")
        sid = lax.axis_index("s")

        zeros = jnp.zeros((16,), jnp.float32)

        def zero_body(i, _):
            r = i // (feat // 16)
            c = i % (feat // 16)
            buf[0, r, pl.ds(c * 16, 16)] = zeros
            return 0

        lax.fori_loop(0, K * (feat // 16), zero_body, 0)
        off = 0
        while off < TROWS:
            rows = min(K, TROWS - off)
            pltpu.sync_copy(
                buf.at[0, pl.ds(0, rows)], acc.at[pl.ds(sid * TROWS + off, rows)]
            )
            off += rows
        plsc.subcore_barrier()

        for stage in range(CH // HCH):
            pltpu.sync_copy(
                src_hbm.at[cid, sid, pl.ds(stage * HCH, HCH)], idx_s
            )
            pltpu.sync_copy(
                dst_hbm.at[cid, sid, pl.ds(stage * HCH, HCH)], idx_d
            )
            for b in range(NBUF - 1):
                pltpu.async_copy(g_hbm.at[idx_s.at[b]], buf.at[b], sem_g)

            def chunk_body(j, _):
                p = j % NBUF
                pltpu.make_async_copy(
                    g_hbm.at[idx_s.at[j]], buf.at[p], sem_g
                ).wait()
                pltpu.async_copy(buf.at[p], acc.at[idx_d.at[j]], sem_s, add=True)

                @pl.when(j >= 1)
                def _():
                    pltpu.make_async_copy(
                        buf.at[(j - 1) % NBUF], acc.at[idx_d.at[j - 1]], sem_s
                    ).wait()

                @pl.when(j + NBUF - 1 < HCH)
                def _():
                    pltpu.async_copy(
                        g_hbm.at[idx_s.at[j + NBUF - 1]],
                        buf.at[(j + NBUF - 1) % NBUF],
                        sem_g,
                    )

                return 0

            lax.fori_loop(0, HCH, chunk_body, 0)
            last = (HCH - 1) % NBUF
            pltpu.make_async_copy(
                buf.at[last], acc.at[idx_d.at[HCH - 1]], sem_s
            ).wait()
        plsc.subcore_barrier()

        off = 0
        while off < TROWS:
            rows = min(K, TROWS - off)
            pltpu.sync_copy(
                acc.at[pl.ds(sid * TROWS + off, rows)],
                out_hbm.at[cid, pl.ds(sid * TROWS + off, rows)],
            )
            off += rows

    return agg_kernel(g, src_r, dst_r)


def _mm_body(x_ref, w_ref, o_ref):
    o_ref[...] = jnp.dot(x_ref[...], w_ref[...], preferred_element_type=jnp.float32)


def _matmul(x, w):
    n, fin = x.shape
    fout = w.shape[1]
    return pl.pallas_call(
        _mm_body,
        grid=(n // BR,),
        in_specs=[
            pl.BlockSpec((BR, fin), lambda i: (i, 0)),
            pl.BlockSpec((fin, fout), lambda i: (0, 0)),
        ],
        out_specs=pl.BlockSpec((BR, fout), lambda i: (i, 0)),
        out_shape=jax.ShapeDtypeStruct((n, fout), jnp.float32),
    )(x, w)


def _combine_mm_body(g_ref, a0_ref, a1_ref, b_ref, w_ref, o_ref):
    h = g_ref[...] + a0_ref[0] + a1_ref[0] + b_ref[...]
    h = jnp.maximum(h, 0.0)
    o_ref[...] = jnp.dot(h, w_ref[...], preferred_element_type=jnp.float32)


def _combine_matmul(g, agg, b, w):
    n, fin = g.shape
    fout = w.shape[1]
    return pl.pallas_call(
        _combine_mm_body,
        grid=(n // BR,),
        in_specs=[
            pl.BlockSpec((BR, fin), lambda i: (i, 0)),
            pl.BlockSpec((1, BR, fin), lambda i: (0, i, 0)),
            pl.BlockSpec((1, BR, fin), lambda i: (1, i, 0)),
            pl.BlockSpec((1, fin), lambda i: (0, 0)),
            pl.BlockSpec((fin, fout), lambda i: (0, 0)),
        ],
        out_specs=pl.BlockSpec((BR, fout), lambda i: (i, 0)),
        out_shape=jax.ShapeDtypeStruct((n, fout), jnp.float32),
    )(g, agg, agg, b.reshape(1, fin), w)


def _combine_relu_body(g_ref, a0_ref, a1_ref, b_ref, o_ref):
    o_ref[...] = jnp.maximum(g_ref[...] + a0_ref[0] + a1_ref[0] + b_ref[...], 0.0)


def _combine_relu(g, agg, b):
    n, f = g.shape
    return pl.pallas_call(
        _combine_relu_body,
        grid=(n // BR,),
        in_specs=[
            pl.BlockSpec((BR, f), lambda i: (i, 0)),
            pl.BlockSpec((1, BR, f), lambda i: (0, i, 0)),
            pl.BlockSpec((1, BR, f), lambda i: (1, i, 0)),
            pl.BlockSpec((1, f), lambda i: (0, 0)),
        ],
        out_specs=pl.BlockSpec((BR, f), lambda i: (i, 0)),
        out_shape=jax.ShapeDtypeStruct((n, f), jnp.float32),
    )(g, agg, agg, b.reshape(1, f))


def _final_mm_body(h_ref, a0_ref, a1_ref, w_ref, b_ref, o_ref):
    hh = h_ref[...] + a0_ref[0] + a1_ref[0]
    o_ref[...] = (
        jnp.dot(hh, w_ref[...], preferred_element_type=jnp.float32) + b_ref[...]
    )


def _final_matmul(h, agg, w, b):
    n, fin = h.shape
    fout = w.shape[1]
    return pl.pallas_call(
        _final_mm_body,
        grid=(n // BR,),
        in_specs=[
            pl.BlockSpec((BR, fin), lambda i: (i, 0)),
            pl.BlockSpec((1, BR, fin), lambda i: (0, i, 0)),
            pl.BlockSpec((1, BR, fin), lambda i: (1, i, 0)),
            pl.BlockSpec((fin, fout), lambda i: (0, 0)),
            pl.BlockSpec((1, fout), lambda i: (0, 0)),
        ],
        out_specs=pl.BlockSpec((BR, fout), lambda i: (i, 0)),
        out_shape=jax.ShapeDtypeStruct((n, fout), jnp.float32),
    )(h, agg, agg, w, b.reshape(1, fout))


def kernel(x, edge_index, W1, b1, W2, b2, W3, b3, W4, b4):
    src = edge_index[0]
    dst = edge_index[1]
    pad = E_PAD - E
    pad_src = (jnp.arange(pad, dtype=jnp.int32) * 997) % N
    pad_dst = N + jnp.arange(pad, dtype=jnp.int32) % (AGG_ROWS - N)
    src_r = jnp.concatenate([src, pad_src]).reshape(NC, NS, CH, K)
    dst_r = jnp.concatenate([dst, pad_dst]).reshape(NC, NS, CH, K)

    g = _matmul(x, W1)
    agg = _sc_aggregate(g, src_r, dst_r, 128)
    g = _combine_matmul(g, agg, b1, W2)
    agg = _sc_aggregate(g, src_r, dst_r, 128)
    g = _combine_matmul(g, agg, b2, W3)
    agg = _sc_aggregate(g, src_r, dst_r, 128)
    h = _combine_relu(g, agg, b3)
    agg = _sc_aggregate(h, src_r, dst_r, 128)
    return _final_matmul(h, agg, W4, b4)


# speedup vs baseline: 2.3823x; 1.0231x over previous
import functools

import jax
import jax.numpy as jnp
from jax import lax
from jax.experimental import pallas as pl
from jax.experimental.pallas import tpu as pltpu
from jax.experimental.pallas import tpu_sc as plsc

N = 10000
E = 320000
NC = 2
NS = 16
K = 64
CH = 160
HCH = CH // 4
NBUF = 4
E_PAD = NC * NS * CH * K
AGG_ROWS = 10112
TROWS = AGG_ROWS // NS
BR = 2000


def _sc_aggregate(g, src_r, dst_r, feat):
    mesh = plsc.VectorSubcoreMesh(
        core_axis_name="c", subcore_axis_name="s", num_cores=NC, num_subcores=NS
    )

    @functools.partial(
        pl.kernel,
        out_type=jax.ShapeDtypeStruct((NC, AGG_ROWS, feat), jnp.float32),
        mesh=mesh,
        scratch_types=[
            pltpu.VMEM((HCH, K), jnp.int32),
            pltpu.VMEM((HCH, K), jnp.int32),
            pltpu.VMEM((NBUF, K, feat), jnp.float32),
            pltpu.VMEM_SHARED((AGG_ROWS, feat), jnp.float32),
            pltpu.SemaphoreType.DMA,
            pltpu.SemaphoreType.DMA,
        ],
    )
    def agg_kernel(
        g_hbm, src_hbm, dst_hbm, out_hbm, idx_s, idx_d, buf, acc, sem_g, sem_s
    ):
        cid = lax.axis_index("c")
        sid = lax.axis_index("s")

        zeros = jnp.zeros((16,), jnp.float32)

        def zero_body(i, _):
            r = i // (feat // 16)
            c = i % (feat // 16)
            buf[0, r, pl.ds(c * 16, 16)] = zeros
            return 0

        lax.fori_loop(0, K * (feat // 16), zero_body, 0)
        off = 0
        while off < TROWS:
            rows = min(K, TROWS - off)
            pltpu.sync_copy(
                buf.at[0, pl.ds(0, rows)], acc.at[pl.ds(sid * TROWS + off, rows)]
            )
            off += rows
        plsc.subcore_barrier()

        for stage in range(CH // HCH):
            pltpu.sync_copy(
                src_hbm.at[cid, sid, pl.ds(stage * HCH, HCH)], idx_s
            )
            pltpu.sync_copy(
                dst_hbm.at[cid, sid, pl.ds(stage * HCH, HCH)], idx_d
            )
            for b in range(NBUF - 1):
                pltpu.async_copy(g_hbm.at[idx_s.at[b]], buf.at[b], sem_g)

            def chunk_body(j, _):
                p = j % NBUF
                pltpu.make_async_copy(
                    g_hbm.at[idx_s.at[j]], buf.at[p], sem_g
                ).wait()
                pltpu.async_copy(buf.at[p], acc.at[idx_d.at[j]], sem_s, add=True)

                @pl.when(j >= 1)
                def _():
                    pltpu.make_async_copy(
                        buf.at[(j - 1) % NBUF], acc.at[idx_d.at[j - 1]], sem_s
                    ).wait()

                @pl.when(j + NBUF - 1 < HCH)
                def _():
                    pltpu.async_copy(
                        g_hbm.at[idx_s.at[j + NBUF - 1]],
                        buf.at[(j + NBUF - 1) % NBUF],
                        sem_g,
                    )

                return 0

            lax.fori_loop(0, HCH, chunk_body, 0)
            last = (HCH - 1) % NBUF
            pltpu.make_async_copy(
                buf.at[last], acc.at[idx_d.at[HCH - 1]], sem_s
            ).wait()
        plsc.subcore_barrier()

        off = 0
        while off < TROWS:
            rows = min(K, TROWS - off)
            pltpu.sync_copy(
                acc.at[pl.ds(sid * TROWS + off, rows)],
                out_hbm.at[cid, pl.ds(sid * TROWS + off, rows)],
            )
            off += rows

    return agg_kernel(g, src_r, dst_r)


def _mm_body(x_ref, w_ref, o_ref):
    o_ref[...] = jnp.dot(
        x_ref[...].astype(jnp.bfloat16),
        w_ref[...].astype(jnp.bfloat16),
        preferred_element_type=jnp.float32,
    )


def _matmul(x, w):
    n, fin = x.shape
    fout = w.shape[1]
    return pl.pallas_call(
        _mm_body,
        grid=(n // BR,),
        in_specs=[
            pl.BlockSpec((BR, fin), lambda i: (i, 0)),
            pl.BlockSpec((fin, fout), lambda i: (0, 0)),
        ],
        out_specs=pl.BlockSpec((BR, fout), lambda i: (i, 0)),
        out_shape=jax.ShapeDtypeStruct((n, fout), jnp.float32),
    )(x, w)


def _combine_mm_body(g_ref, a0_ref, a1_ref, b_ref, w_ref, o_ref):
    h = g_ref[...] + a0_ref[0] + a1_ref[0] + b_ref[...]
    h = jnp.maximum(h, 0.0)
    o_ref[...] = jnp.dot(
        h.astype(jnp.bfloat16),
        w_ref[...].astype(jnp.bfloat16),
        preferred_element_type=jnp.float32,
    )


def _combine_matmul(g, agg, b, w):
    n, fin = g.shape
    fout = w.shape[1]
    return pl.pallas_call(
        _combine_mm_body,
        grid=(n // BR,),
        in_specs=[
            pl.BlockSpec((BR, fin), lambda i: (i, 0)),
            pl.BlockSpec((1, BR, fin), lambda i: (0, i, 0)),
            pl.BlockSpec((1, BR, fin), lambda i: (1, i, 0)),
            pl.BlockSpec((1, fin), lambda i: (0, 0)),
            pl.BlockSpec((fin, fout), lambda i: (0, 0)),
        ],
        out_specs=pl.BlockSpec((BR, fout), lambda i: (i, 0)),
        out_shape=jax.ShapeDtypeStruct((n, fout), jnp.float32),
    )(g, agg, agg, b.reshape(1, fin), w)


def _combine_relu_body(g_ref, a0_ref, a1_ref, b_ref, o_ref):
    o_ref[...] = jnp.maximum(g_ref[...] + a0_ref[0] + a1_ref[0] + b_ref[...], 0.0)


def _combine_relu(g, agg, b):
    n, f = g.shape
    return pl.pallas_call(
        _combine_relu_body,
        grid=(n // BR,),
        in_specs=[
            pl.BlockSpec((BR, f), lambda i: (i, 0)),
            pl.BlockSpec((1, BR, f), lambda i: (0, i, 0)),
            pl.BlockSpec((1, BR, f), lambda i: (1, i, 0)),
            pl.BlockSpec((1, f), lambda i: (0, 0)),
        ],
        out_specs=pl.BlockSpec((BR, f), lambda i: (i, 0)),
        out_shape=jax.ShapeDtypeStruct((n, f), jnp.float32),
    )(g, agg, agg, b.reshape(1, f))


def _final_mm_body(h_ref, a0_ref, a1_ref, w_ref, b_ref, o_ref):
    hh = h_ref[...] + a0_ref[0] + a1_ref[0]
    o_ref[...] = (
        jnp.dot(
            hh.astype(jnp.bfloat16),
            w_ref[...].astype(jnp.bfloat16),
            preferred_element_type=jnp.float32,
        )
        + b_ref[...]
    )


def _final_matmul(h, agg, w, b):
    n, fin = h.shape
    fout = w.shape[1]
    return pl.pallas_call(
        _final_mm_body,
        grid=(n // BR,),
        in_specs=[
            pl.BlockSpec((BR, fin), lambda i: (i, 0)),
            pl.BlockSpec((1, BR, fin), lambda i: (0, i, 0)),
            pl.BlockSpec((1, BR, fin), lambda i: (1, i, 0)),
            pl.BlockSpec((fin, fout), lambda i: (0, 0)),
            pl.BlockSpec((1, fout), lambda i: (0, 0)),
        ],
        out_specs=pl.BlockSpec((BR, fout), lambda i: (i, 0)),
        out_shape=jax.ShapeDtypeStruct((n, fout), jnp.float32),
    )(h, agg, agg, w, b.reshape(1, fout))


def kernel(x, edge_index, W1, b1, W2, b2, W3, b3, W4, b4):
    src = edge_index[0]
    dst = edge_index[1]
    pad = E_PAD - E
    pad_src = (jnp.arange(pad, dtype=jnp.int32) * 997) % N
    pad_dst = N + jnp.arange(pad, dtype=jnp.int32) % (AGG_ROWS - N)
    src_r = jnp.concatenate([src, pad_src]).reshape(NC, NS, CH, K)
    dst_r = jnp.concatenate([dst, pad_dst]).reshape(NC, NS, CH, K)

    g = _matmul(x, W1)
    agg = _sc_aggregate(g, src_r, dst_r, 128)
    g = _combine_matmul(g, agg, b1, W2)
    agg = _sc_aggregate(g, src_r, dst_r, 128)
    g = _combine_matmul(g, agg, b2, W3)
    agg = _sc_aggregate(g, src_r, dst_r, 128)
    h = _combine_relu(g, agg, b3)
    agg = _sc_aggregate(h, src_r, dst_r, 128)
    return _final_matmul(h, agg, W4, b4)


# speedup vs baseline: 2.4389x; 1.0237x over previous
import functools

import jax
import jax.numpy as jnp
from jax import lax
from jax.experimental import pallas as pl
from jax.experimental.pallas import tpu as pltpu
from jax.experimental.pallas import tpu_sc as plsc

N = 10000
E = 320000
NC = 2
NS = 16
K = 64
CH = 160
SCH = 32
NST = CH // SCH
NBUF = 4
E_PAD = NC * NS * CH * K
AGG_ROWS = 10112
TROWS = AGG_ROWS // NS
BR = 2000


def _sc_aggregate(g, src_r, dst_r, feat):
    mesh = plsc.VectorSubcoreMesh(
        core_axis_name="c", subcore_axis_name="s", num_cores=NC, num_subcores=NS
    )

    @functools.partial(
        pl.kernel,
        out_type=jax.ShapeDtypeStruct((NC, AGG_ROWS, feat), jnp.float32),
        mesh=mesh,
        scratch_types=[
            pltpu.VMEM((2, SCH, K), jnp.int32),
            pltpu.VMEM((2, SCH, K), jnp.int32),
            pltpu.VMEM((NBUF, K, feat), jnp.float32),
            pltpu.VMEM_SHARED((AGG_ROWS, feat), jnp.float32),
            pltpu.SemaphoreType.DMA,
            pltpu.SemaphoreType.DMA,
        ],
    )
    def agg_kernel(
        g_hbm, src_hbm, dst_hbm, out_hbm, idx_s, idx_d, buf, acc, sem_g, sem_s
    ):
        cid = lax.axis_index("c")
        sid = lax.axis_index("s")

        zeros = jnp.zeros((16,), jnp.float32)

        def zero_body(i, _):
            r = i // (feat // 16)
            c = i % (feat // 16)
            buf[0, r, pl.ds(c * 16, 16)] = zeros
            return 0

        lax.fori_loop(0, K * (feat // 16), zero_body, 0)
        off = 0
        while off < TROWS:
            rows = min(K, TROWS - off)
            pltpu.sync_copy(
                buf.at[0, pl.ds(0, rows)], acc.at[pl.ds(sid * TROWS + off, rows)]
            )
            off += rows
        plsc.subcore_barrier()

        pltpu.sync_copy(src_hbm.at[cid, sid, pl.ds(0, SCH)], idx_s.at[0])
        pltpu.sync_copy(dst_hbm.at[cid, sid, pl.ds(0, SCH)], idx_d.at[0])
        for b in range(NBUF - 1):
            pltpu.async_copy(g_hbm.at[idx_s.at[0, b]], buf.at[b], sem_g)

        def chunk_body(j, _):
            st = j // SCH
            sp = st % 2
            jj = j % SCH
            p = j % NBUF
            pltpu.make_async_copy(
                g_hbm.at[idx_s.at[sp, jj]], buf.at[p], sem_g
            ).wait()
            pltpu.async_copy(buf.at[p], acc.at[idx_d.at[sp, jj]], sem_s, add=True)

            @pl.when(j >= 1)
            def _():
                jm = j - 1
                pltpu.make_async_copy(
                    buf.at[jm % NBUF],
                    acc.at[idx_d.at[(jm // SCH) % 2, jm % SCH]],
                    sem_s,
                ).wait()

            @pl.when(jnp.logical_and(jj == 8, st + 1 < NST))
            def _():
                base = pl.multiple_of((st + 1) * SCH, SCH)
                pltpu.sync_copy(
                    src_hbm.at[cid, sid, pl.ds(base, SCH)], idx_s.at[(st + 1) % 2]
                )
                pltpu.sync_copy(
                    dst_hbm.at[cid, sid, pl.ds(base, SCH)], idx_d.at[(st + 1) % 2]
                )

            @pl.when(j + NBUF - 1 < CH)
            def _():
                jn = j + NBUF - 1
                pltpu.async_copy(
                    g_hbm.at[idx_s.at[(jn // SCH) % 2, jn % SCH]],
                    buf.at[jn % NBUF],
                    sem_g,
                )

            return 0

        lax.fori_loop(0, CH, chunk_body, 0)
        jm = CH - 1
        pltpu.make_async_copy(
            buf.at[jm % NBUF],
            acc.at[idx_d.at[(jm // SCH) % 2, jm % SCH]],
            sem_s,
        ).wait()
        plsc.subcore_barrier()

        off = 0
        while off < TROWS:
            rows = min(K, TROWS - off)
            pltpu.sync_copy(
                acc.at[pl.ds(sid * TROWS + off, rows)],
                out_hbm.at[cid, pl.ds(sid * TROWS + off, rows)],
            )
            off += rows

    return agg_kernel(g, src_r, dst_r)


def _mm_body(x_ref, w_ref, o_ref):
    o_ref[...] = jnp.dot(
        x_ref[...].astype(jnp.bfloat16),
        w_ref[...].astype(jnp.bfloat16),
        preferred_element_type=jnp.float32,
    )


def _matmul(x, w):
    n, fin = x.shape
    fout = w.shape[1]
    return pl.pallas_call(
        _mm_body,
        grid=(n // BR,),
        in_specs=[
            pl.BlockSpec((BR, fin), lambda i: (i, 0)),
            pl.BlockSpec((fin, fout), lambda i: (0, 0)),
        ],
        out_specs=pl.BlockSpec((BR, fout), lambda i: (i, 0)),
        out_shape=jax.ShapeDtypeStruct((n, fout), jnp.float32),
    )(x, w)


def _combine_mm_body(g_ref, a0_ref, a1_ref, b_ref, w_ref, o_ref):
    h = g_ref[...] + a0_ref[0] + a1_ref[0] + b_ref[...]
    h = jnp.maximum(h, 0.0)
    o_ref[...] = jnp.dot(
        h.astype(jnp.bfloat16),
        w_ref[...].astype(jnp.bfloat16),
        preferred_element_type=jnp.float32,
    )


def _combine_matmul(g, agg, b, w):
    n, fin = g.shape
    fout = w.shape[1]
    return pl.pallas_call(
        _combine_mm_body,
        grid=(n // BR,),
        in_specs=[
            pl.BlockSpec((BR, fin), lambda i: (i, 0)),
            pl.BlockSpec((1, BR, fin), lambda i: (0, i, 0)),
            pl.BlockSpec((1, BR, fin), lambda i: (1, i, 0)),
            pl.BlockSpec((1, fin), lambda i: (0, 0)),
            pl.BlockSpec((fin, fout), lambda i: (0, 0)),
        ],
        out_specs=pl.BlockSpec((BR, fout), lambda i: (i, 0)),
        out_shape=jax.ShapeDtypeStruct((n, fout), jnp.float32),
    )(g, agg, agg, b.reshape(1, fin), w)


def _combine_relu_body(g_ref, a0_ref, a1_ref, b_ref, o_ref):
    o_ref[...] = jnp.maximum(g_ref[...] + a0_ref[0] + a1_ref[0] + b_ref[...], 0.0)


def _combine_relu(g, agg, b):
    n, f = g.shape
    return pl.pallas_call(
        _combine_relu_body,
        grid=(n // BR,),
        in_specs=[
            pl.BlockSpec((BR, f), lambda i: (i, 0)),
            pl.BlockSpec((1, BR, f), lambda i: (0, i, 0)),
            pl.BlockSpec((1, BR, f), lambda i: (1, i, 0)),
            pl.BlockSpec((1, f), lambda i: (0, 0)),
        ],
        out_specs=pl.BlockSpec((BR, f), lambda i: (i, 0)),
        out_shape=jax.ShapeDtypeStruct((n, f), jnp.float32),
    )(g, agg, agg, b.reshape(1, f))


def _final_mm_body(h_ref, a0_ref, a1_ref, w_ref, b_ref, o_ref):
    hh = h_ref[...] + a0_ref[0] + a1_ref[0]
    o_ref[...] = (
        jnp.dot(
            hh.astype(jnp.bfloat16),
            w_ref[...].astype(jnp.bfloat16),
            preferred_element_type=jnp.float32,
        )
        + b_ref[...]
    )


def _final_matmul(h, agg, w, b):
    n, fin = h.shape
    fout = w.shape[1]
    return pl.pallas_call(
        _final_mm_body,
        grid=(n // BR,),
        in_specs=[
            pl.BlockSpec((BR, fin), lambda i: (i, 0)),
            pl.BlockSpec((1, BR, fin), lambda i: (0, i, 0)),
            pl.BlockSpec((1, BR, fin), lambda i: (1, i, 0)),
            pl.BlockSpec((fin, fout), lambda i: (0, 0)),
            pl.BlockSpec((1, fout), lambda i: (0, 0)),
        ],
        out_specs=pl.BlockSpec((BR, fout), lambda i: (i, 0)),
        out_shape=jax.ShapeDtypeStruct((n, fout), jnp.float32),
    )(h, agg, agg, w, b.reshape(1, fout))


def kernel(x, edge_index, W1, b1, W2, b2, W3, b3, W4, b4):
    src = edge_index[0]
    dst = edge_index[1]
    pad = E_PAD - E
    pad_src = (jnp.arange(pad, dtype=jnp.int32) * 997) % N
    pad_dst = N + jnp.arange(pad, dtype=jnp.int32) % (AGG_ROWS - N)
    src_r = jnp.concatenate([src, pad_src]).reshape(NC, NS, CH, K)
    dst_r = jnp.concatenate([dst, pad_dst]).reshape(NC, NS, CH, K)

    g = _matmul(x, W1)
    agg = _sc_aggregate(g, src_r, dst_r, 128)
    g = _combine_matmul(g, agg, b1, W2)
    agg = _sc_aggregate(g, src_r, dst_r, 128)
    g = _combine_matmul(g, agg, b2, W3)
    agg = _sc_aggregate(g, src_r, dst_r, 128)
    h = _combine_relu(g, agg, b3)
    agg = _sc_aggregate(h, src_r, dst_r, 128)
    return _final_matmul(h, agg, W4, b4)


# speedup vs baseline: 2.4793x; 1.0166x over previous
import functools

import jax
import jax.numpy as jnp
from jax import lax
from jax.experimental import pallas as pl
from jax.experimental.pallas import tpu as pltpu
from jax.experimental.pallas import tpu_sc as plsc

N = 10000
E = 320000
NC = 2
NS = 16
K = 64
CH = 160
SCH = 32
NST = CH // SCH
NBUF = 4
E_PAD = NC * NS * CH * K
AGG_ROWS = 10112
TROWS = AGG_ROWS // NS
BR = 2000


def _sc_aggregate(g, src_r, dst_r, feat):
    mesh = plsc.VectorSubcoreMesh(
        core_axis_name="c", subcore_axis_name="s", num_cores=NC, num_subcores=NS
    )

    @functools.partial(
        pl.kernel,
        out_type=jax.ShapeDtypeStruct((NC, AGG_ROWS, feat), jnp.float32),
        mesh=mesh,
        scratch_types=[
            pltpu.VMEM((2, SCH, K), jnp.int32),
            pltpu.VMEM((2, SCH, K), jnp.int32),
            pltpu.VMEM((NBUF, K, feat), jnp.float32),
            pltpu.VMEM_SHARED((AGG_ROWS, feat), jnp.float32),
            pltpu.SemaphoreType.DMA,
            pltpu.SemaphoreType.DMA,
        ],
    )
    def agg_kernel(
        g_hbm, src_hbm, dst_hbm, out_hbm, idx_s, idx_d, buf, acc, sem_g, sem_s
    ):
        cid = lax.axis_index("c")
        sid = lax.axis_index("s")

        pltpu.sync_copy(src_hbm.at[cid, sid, pl.ds(0, SCH)], idx_s.at[0])
        pltpu.sync_copy(dst_hbm.at[cid, sid, pl.ds(0, SCH)], idx_d.at[0])
        for b in range(NBUF - 1):
            pltpu.async_copy(g_hbm.at[idx_s.at[0, b]], buf.at[b], sem_g)

        zeros = jnp.zeros((16,), jnp.float32)

        def zero_body(i, _):
            r = i // (feat // 16)
            c = i % (feat // 16)
            buf[NBUF - 1, r, pl.ds(c * 16, 16)] = zeros
            return 0

        lax.fori_loop(0, K * (feat // 16), zero_body, 0)
        off = 0
        while off < TROWS:
            rows = min(K, TROWS - off)
            pltpu.sync_copy(
                buf.at[NBUF - 1, pl.ds(0, rows)],
                acc.at[pl.ds(sid * TROWS + off, rows)],
            )
            off += rows
        plsc.subcore_barrier()

        def chunk_body(j, _):
            st = j // SCH
            sp = st % 2
            jj = j % SCH
            p = j % NBUF
            pltpu.make_async_copy(
                g_hbm.at[idx_s.at[sp, jj]], buf.at[p], sem_g
            ).wait()
            pltpu.async_copy(buf.at[p], acc.at[idx_d.at[sp, jj]], sem_s, add=True)

            @pl.when(j >= 1)
            def _():
                jm = j - 1
                pltpu.make_async_copy(
                    buf.at[jm % NBUF],
                    acc.at[idx_d.at[(jm // SCH) % 2, jm % SCH]],
                    sem_s,
                ).wait()

            @pl.when(jnp.logical_and(jj == 8, st + 1 < NST))
            def _():
                base = pl.multiple_of((st + 1) * SCH, SCH)
                pltpu.sync_copy(
                    src_hbm.at[cid, sid, pl.ds(base, SCH)], idx_s.at[(st + 1) % 2]
                )
                pltpu.sync_copy(
                    dst_hbm.at[cid, sid, pl.ds(base, SCH)], idx_d.at[(st + 1) % 2]
                )

            @pl.when(j + NBUF - 1 < CH)
            def _():
                jn = j + NBUF - 1
                pltpu.async_copy(
                    g_hbm.at[idx_s.at[(jn // SCH) % 2, jn % SCH]],
                    buf.at[jn % NBUF],
                    sem_g,
                )

            return 0

        lax.fori_loop(0, CH, chunk_body, 0)
        jm = CH - 1
        pltpu.make_async_copy(
            buf.at[jm % NBUF],
            acc.at[idx_d.at[(jm // SCH) % 2, jm % SCH]],
            sem_s,
        ).wait()
        plsc.subcore_barrier()

        pltpu.sync_copy(
            acc.at[pl.ds(sid * TROWS, TROWS)],
            out_hbm.at[cid, pl.ds(sid * TROWS, TROWS)],
        )

    return agg_kernel(g, src_r, dst_r)


def _mm_body(x_ref, w_ref, o_ref):
    o_ref[...] = jnp.dot(
        x_ref[...].astype(jnp.bfloat16),
        w_ref[...].astype(jnp.bfloat16),
        preferred_element_type=jnp.float32,
    )


def _matmul(x, w):
    n, fin = x.shape
    fout = w.shape[1]
    return pl.pallas_call(
        _mm_body,
        grid=(n // BR,),
        in_specs=[
            pl.BlockSpec((BR, fin), lambda i: (i, 0)),
            pl.BlockSpec((fin, fout), lambda i: (0, 0)),
        ],
        out_specs=pl.BlockSpec((BR, fout), lambda i: (i, 0)),
        out_shape=jax.ShapeDtypeStruct((n, fout), jnp.float32),
    )(x, w)


def _combine_mm_body(g_ref, a0_ref, a1_ref, b_ref, w_ref, o_ref):
    h = g_ref[...] + a0_ref[0] + a1_ref[0] + b_ref[...]
    h = jnp.maximum(h, 0.0)
    o_ref[...] = jnp.dot(
        h.astype(jnp.bfloat16),
        w_ref[...].astype(jnp.bfloat16),
        preferred_element_type=jnp.float32,
    )


def _combine_matmul(g, agg, b, w):
    n, fin = g.shape
    fout = w.shape[1]
    return pl.pallas_call(
        _combine_mm_body,
        grid=(n // BR,),
        in_specs=[
            pl.BlockSpec((BR, fin), lambda i: (i, 0)),
            pl.BlockSpec((1, BR, fin), lambda i: (0, i, 0)),
            pl.BlockSpec((1, BR, fin), lambda i: (1, i, 0)),
            pl.BlockSpec((1, fin), lambda i: (0, 0)),
            pl.BlockSpec((fin, fout), lambda i: (0, 0)),
        ],
        out_specs=pl.BlockSpec((BR, fout), lambda i: (i, 0)),
        out_shape=jax.ShapeDtypeStruct((n, fout), jnp.float32),
    )(g, agg, agg, b.reshape(1, fin), w)


def _combine_relu_body(g_ref, a0_ref, a1_ref, b_ref, o_ref):
    o_ref[...] = jnp.maximum(g_ref[...] + a0_ref[0] + a1_ref[0] + b_ref[...], 0.0)


def _combine_relu(g, agg, b):
    n, f = g.shape
    return pl.pallas_call(
        _combine_relu_body,
        grid=(n // BR,),
        in_specs=[
            pl.BlockSpec((BR, f), lambda i: (i, 0)),
            pl.BlockSpec((1, BR, f), lambda i: (0, i, 0)),
            pl.BlockSpec((1, BR, f), lambda i: (1, i, 0)),
            pl.BlockSpec((1, f), lambda i: (0, 0)),
        ],
        out_specs=pl.BlockSpec((BR, f), lambda i: (i, 0)),
        out_shape=jax.ShapeDtypeStruct((n, f), jnp.float32),
    )(g, agg, agg, b.reshape(1, f))


def _final_mm_body(h_ref, a0_ref, a1_ref, w_ref, b_ref, o_ref):
    hh = h_ref[...] + a0_ref[0] + a1_ref[0]
    o_ref[...] = (
        jnp.dot(
            hh.astype(jnp.bfloat16),
            w_ref[...].astype(jnp.bfloat16),
            preferred_element_type=jnp.float32,
        )
        + b_ref[...]
    )


def _final_matmul(h, agg, w, b):
    n, fin = h.shape
    fout = w.shape[1]
    return pl.pallas_call(
        _final_mm_body,
        grid=(n // BR,),
        in_specs=[
            pl.BlockSpec((BR, fin), lambda i: (i, 0)),
            pl.BlockSpec((1, BR, fin), lambda i: (0, i, 0)),
            pl.BlockSpec((1, BR, fin), lambda i: (1, i, 0)),
            pl.BlockSpec((fin, fout), lambda i: (0, 0)),
            pl.BlockSpec((1, fout), lambda i: (0, 0)),
        ],
        out_specs=pl.BlockSpec((BR, fout), lambda i: (i, 0)),
        out_shape=jax.ShapeDtypeStruct((n, fout), jnp.float32),
    )(h, agg, agg, w, b.reshape(1, fout))


def kernel(x, edge_index, W1, b1, W2, b2, W3, b3, W4, b4):
    src = edge_index[0]
    dst = edge_index[1]
    pad = E_PAD - E
    pad_src = (jnp.arange(pad, dtype=jnp.int32) * 997) % N
    pad_dst = N + jnp.arange(pad, dtype=jnp.int32) % (AGG_ROWS - N)
    src_r = jnp.concatenate([src, pad_src]).reshape(NC, NS, CH, K)
    dst_r = jnp.concatenate([dst, pad_dst]).reshape(NC, NS, CH, K)

    g = _matmul(x, W1)
    agg = _sc_aggregate(g, src_r, dst_r, 128)
    g = _combine_matmul(g, agg, b1, W2)
    agg = _sc_aggregate(g, src_r, dst_r, 128)
    g = _combine_matmul(g, agg, b2, W3)
    agg = _sc_aggregate(g, src_r, dst_r, 128)
    h = _combine_relu(g, agg, b3)
    agg = _sc_aggregate(h, src_r, dst_r, 128)
    return _final_matmul(h, agg, W4, b4)
